# baseline (speedup 1.0000x reference)
import jax
import jax.numpy as jnp
from jax import lax
from jax.experimental import pallas as pl
from jax.experimental.pallas import tpu as pltpu
from jax.experimental.pallas import tpu_sc as plsc

B, S, D = 4, 8192, 1024
NC, NS = 2, 16
NW = NC * NS
POS_W = S // NW
R = 16
BLK = R * D
NBLK = POS_W // R
UNROLL = 8


def _body(x_hbm, w_hbm, out_hbm, wbuf, xbuf):
    wid = lax.axis_index("s") * NC + lax.axis_index("c")
    base = wid * (POS_W * D)

    def do_block(j, carry):
        off = base + j * BLK
        pltpu.sync_copy(w_hbm.at[pl.ds(off, BLK)], wbuf)
        for b in range(B):
            xoff = b * (S * D) + off
            pltpu.sync_copy(x_hbm.at[pl.ds(xoff, BLK)], xbuf)

            def add_chunk(i, c2):
                o = i * (16 * UNROLL)
                for u in range(UNROLL):
                    sl = pl.ds(o + u * 16, 16)
                    xbuf[sl] = xbuf[sl] + wbuf[sl]
                return c2

            lax.fori_loop(0, BLK // (16 * UNROLL), add_chunk, 0)
            pltpu.sync_copy(xbuf, out_hbm.at[pl.ds(xoff, BLK)])
        return carry

    lax.fori_loop(0, NBLK, do_block, 0)


@jax.jit
def _pe_add(x_flat, w_flat):
    mesh = plsc.VectorSubcoreMesh(core_axis_name="c", subcore_axis_name="s")
    f = pl.kernel(
        _body,
        out_type=jax.ShapeDtypeStruct((B * S * D,), jnp.float32),
        mesh=mesh,
        scratch_types=[
            pltpu.VMEM((BLK,), jnp.float32),
            pltpu.VMEM((BLK,), jnp.float32),
        ],
    )
    return f(x_flat, w_flat)


def kernel(x, weight):
    out = _pe_add(x.reshape(-1), weight.reshape(-1))
    return out.reshape(x.shape)


# speedup vs baseline: 1.2974x; 1.2974x over previous
import jax
import jax.numpy as jnp
from jax import lax
from jax.experimental import pallas as pl
from jax.experimental.pallas import tpu as pltpu
from jax.experimental.pallas import tpu_sc as plsc

B, S, D = 4, 8192, 1024
NC, NS = 2, 16
NW = NC * NS
POS_W = S // NW
R = 8
BLK = R * D
NBLK = POS_W // R
UNROLL = 4


def _body(x_hbm, w_hbm, out_hbm, *scr):
    wb = [scr[0], scr[5]]
    xb = [scr[1:5], scr[6:10]]
    in_sem = [scr[10], scr[11]]
    out_sem = [scr[12], scr[13]]

    wid = lax.axis_index("s") * NC + lax.axis_index("c")
    base = wid * (POS_W * D)

    def start_in(s, j):
        off = base + j * BLK
        ds_ = [pltpu.async_copy(w_hbm.at[pl.ds(off, BLK)], wb[s], in_sem[s])]
        for b in range(B):
            ds_.append(pltpu.async_copy(
                x_hbm.at[pl.ds(b * (S * D) + off, BLK)], xb[s][b], in_sem[s]))
        return ds_

    def start_out(s, j):
        off = base + j * BLK
        return [pltpu.async_copy(
            xb[s][b], out_hbm.at[pl.ds(b * (S * D) + off, BLK)], out_sem[s])
            for b in range(B)]

    def compute(s):
        x0, x1, x2, x3 = xb[s]
        w = wb[s]

        def add_chunk(i, c2):
            o = i * (16 * UNROLL)
            for u in range(UNROLL):
                sl = pl.ds(o + u * 16, 16)
                wv = w[sl]
                x0[sl] = x0[sl] + wv
                x1[sl] = x1[sl] + wv
                x2[sl] = x2[sl] + wv
                x3[sl] = x3[sl] + wv
            return c2

        lax.fori_loop(0, BLK // (16 * UNROLL), add_chunk, 0)

    in_d = [None, None]
    out_d = [None, None]
    in_d[0] = start_in(0, 0)
    for j in range(NBLK):
        s = j & 1
        if j + 1 < NBLK:
            s2 = (j + 1) & 1
            if out_d[s2] is not None:
                for d in out_d[s2]:
                    d.wait()
            in_d[s2] = start_in(s2, j + 1)
        for d in in_d[s]:
            d.wait()
        compute(s)
        out_d[s] = start_out(s, j)
    for s in (0, 1):
        if out_d[s] is not None:
            for d in out_d[s]:
                d.wait()


@jax.jit
def _pe_add(x_flat, w_flat):
    mesh = plsc.VectorSubcoreMesh(core_axis_name="c", subcore_axis_name="s")
    buf_set = [pltpu.VMEM((BLK,), jnp.float32) for _ in range(1 + B)]
    f = pl.kernel(
        _body,
        out_type=jax.ShapeDtypeStruct((B * S * D,), jnp.float32),
        mesh=mesh,
        scratch_types=buf_set + buf_set + [pltpu.SemaphoreType.DMA] * 4,
    )
    return f(x_flat, w_flat)


def kernel(x, weight):
    out = _pe_add(x.reshape(-1), weight.reshape(-1))
    return out.reshape(x.shape)


# speedup vs baseline: 3.2908x; 2.5364x over previous
import jax
import jax.numpy as jnp
from jax import lax
from jax.experimental import pallas as pl
from jax.experimental.pallas import tpu as pltpu
from jax.experimental.pallas import tpu_sc as plsc

B, S, D = 4, 8192, 1024
NC, NS = 2, 16
NW = NC * NS
POS_W = S // NW
R = 8
BLK = R * D
NBLK = POS_W // R
UNROLL = 4


def _body(x_hbm, w_hbm, out_hbm, *scr):
    wb = [scr[0], scr[5]]
    xb = [scr[1:5], scr[6:10]]
    in_sem = [scr[10], scr[11]]
    out_sem = [scr[12], scr[13]]

    wid = lax.axis_index("s") * NC + lax.axis_index("c")
    base = wid * POS_W

    def start_in(s, j):
        r0 = base + j * R
        pltpu.async_copy(w_hbm.at[pl.ds(r0, R)], wb[s], in_sem[s])
        for b in range(B):
            pltpu.async_copy(
                x_hbm.at[pl.ds(b * S + r0, R)], xb[s][b], in_sem[s])

    def start_out(s, j):
        r0 = base + j * R
        for b in range(B):
            pltpu.async_copy(
                xb[s][b], out_hbm.at[pl.ds(b * S + r0, R)], out_sem[s])

    def wait_in(s):
        pltpu.make_async_copy(w_hbm.at[pl.ds(0, R)], wb[s], in_sem[s]).wait()
        for b in range(B):
            pltpu.make_async_copy(
                x_hbm.at[pl.ds(0, R)], xb[s][b], in_sem[s]).wait()

    def wait_out(s):
        for b in range(B):
            pltpu.make_async_copy(
                xb[s][b], out_hbm.at[pl.ds(0, R)], out_sem[s]).wait()

    def compute(s):
        x0, x1, x2, x3 = xb[s]
        w = wb[s]

        def add_chunk(i, c2):
            o = i * (16 * UNROLL)
            for u in range(UNROLL):
                sl = pl.ds(o + u * 16, 16)
                for r in range(R):
                    wv = w[r, sl]
                    x0[r, sl] = x0[r, sl] + wv
                    x1[r, sl] = x1[r, sl] + wv
                    x2[r, sl] = x2[r, sl] + wv
                    x3[r, sl] = x3[r, sl] + wv
            return c2

        lax.fori_loop(0, D // (16 * UNROLL), add_chunk, 0)

    def process(j, s, has_next, has_prev_out):
        if has_next:
            if has_prev_out:
                wait_out(1 - s)
            start_in(1 - s, j + 1)
        wait_in(s)
        compute(s)
        start_out(s, j)

    start_in(0, 0)
    process(0, 0, True, False)

    def middle(t, c):
        j = 1 + 2 * t
        process(j, 1, True, True)
        process(j + 1, 0, True, True)
        return c

    lax.fori_loop(0, (NBLK - 2) // 2, middle, 0)
    process(NBLK - 1, 1, False, True)
    wait_out(0)
    wait_out(1)


@jax.jit
def _pe_add(x2, w2):
    mesh = plsc.VectorSubcoreMesh(core_axis_name="c", subcore_axis_name="s")
    buf_set = [pltpu.VMEM((R, D), jnp.float32) for _ in range(1 + B)]
    f = pl.kernel(
        _body,
        out_type=jax.ShapeDtypeStruct((B * S, D), jnp.float32),
        mesh=mesh,
        scratch_types=buf_set + buf_set + [pltpu.SemaphoreType.DMA] * 4,
        compiler_params=pltpu.CompilerParams(use_tc_tiling_on_sc=True),
    )
    return f(x2, w2)


def kernel(x, weight):
    out = _pe_add(x.reshape(B * S, D), weight.reshape(S, D))
    return out.reshape(x.shape)
